# baseline (speedup 1.0000x reference)
import functools

import jax
import jax.numpy as jnp
from jax import lax
from jax.experimental import pallas as pl
from jax.experimental.pallas import tpu as pltpu
from jax.experimental.pallas import tpu_sc as plsc

L = 16
NC, NS = 2, 16
NW = NC * NS
C = 4096
ROWS = 4 * 4096
ROWS_PER_W = ROWS // NW
CHUNK = 4
NCHUNK = ROWS_PER_W // CHUNK
NB = 3
NGROUP = (NCHUNK + NB - 1) // NB

_mesh = plsc.VectorSubcoreMesh(
    core_axis_name="c", subcore_axis_name="s", num_cores=NC, num_subcores=NS
)


@functools.partial(
    pl.kernel,
    out_type=jax.ShapeDtypeStruct((ROWS, C), jnp.float32),
    mesh=_mesh,
    scratch_types=[
        pltpu.VMEM((C,), jnp.int32),
        pltpu.VMEM((CHUNK, C), jnp.float32),
        pltpu.VMEM((CHUNK, C), jnp.float32),
        pltpu.VMEM((CHUNK, C), jnp.float32),
        pltpu.VMEM((CHUNK, C), jnp.float32),
        pltpu.VMEM((CHUNK, C), jnp.float32),
        pltpu.VMEM((CHUNK, C), jnp.float32),
        pltpu.SemaphoreType.DMA,
        pltpu.SemaphoreType.DMA,
        pltpu.SemaphoreType.DMA,
        pltpu.SemaphoreType.DMA,
        pltpu.SemaphoreType.DMA,
        pltpu.SemaphoreType.DMA,
        pltpu.SemaphoreType.DMA,
        pltpu.SemaphoreType.DMA,
        pltpu.SemaphoreType.DMA,
        pltpu.SemaphoreType.DMA,
        pltpu.SemaphoreType.DMA,
        pltpu.SemaphoreType.DMA,
    ],
    compiler_params=pltpu.CompilerParams(needs_layout_passes=False),
)
def _shuffle_sc(x_hbm, idx_hbm, out_hbm, idx_v, in0, in1, in2,
                out0, out1, out2, isem0, isem1, isem2, osem0, osem1, osem2,
                isem0b, isem1b, isem2b, osem0b, osem1b, osem2b):
    wid = lax.axis_index("s") * NC + lax.axis_index("c")
    base = wid * ROWS_PER_W

    ins = (in0, in1, in2)
    outs = (out0, out1, out2)
    isems = (isem0, isem1, isem2)
    osems = (osem0, osem1, osem2)
    isems2 = (isem0b, isem1b, isem2b)
    osems2 = (osem0b, osem1b, osem2b)
    H = CHUNK // 2

    pltpu.sync_copy(idx_hbm, idx_v)

    def start_in(g, b):
        row0 = base + g * CHUNK
        pltpu.async_copy(
            x_hbm.at[pl.ds(row0, H), :], ins[b].at[pl.ds(0, H), :], isems[b])
        pltpu.async_copy(
            x_hbm.at[pl.ds(row0 + H, H), :], ins[b].at[pl.ds(H, H), :],
            isems2[b])

    def wait_in(g, b):
        row0 = base + g * CHUNK
        pltpu.make_async_copy(
            x_hbm.at[pl.ds(row0, H), :], ins[b].at[pl.ds(0, H), :], isems[b]
        ).wait()
        pltpu.make_async_copy(
            x_hbm.at[pl.ds(row0 + H, H), :], ins[b].at[pl.ds(H, H), :],
            isems2[b]
        ).wait()

    def start_out(g, b):
        row0 = base + g * CHUNK
        pltpu.async_copy(
            outs[b].at[pl.ds(0, H), :], out_hbm.at[pl.ds(row0, H), :],
            osems[b])
        pltpu.async_copy(
            outs[b].at[pl.ds(H, H), :], out_hbm.at[pl.ds(row0 + H, H), :],
            osems2[b])

    def wait_out(g, b):
        row0 = base + g * CHUNK
        pltpu.make_async_copy(
            outs[b].at[pl.ds(0, H), :], out_hbm.at[pl.ds(row0, H), :],
            osems[b]
        ).wait()
        pltpu.make_async_copy(
            outs[b].at[pl.ds(H, H), :], out_hbm.at[pl.ds(row0 + H, H), :],
            osems2[b]
        ).wait()

    def compute(b):
        in_v = ins[b]
        out_v = outs[b]

        @plsc.parallel_loop(0, C // L, unroll=8)
        def _(j):
            col = j * L
            idxv = idx_v[pl.ds(col, L)]
            for r in range(CHUNK):
                rvec = jnp.full((L,), r, jnp.int32)
                out_v[r, pl.ds(col, L)] = plsc.load_gather(in_v, [rvec, idxv])

    for b in range(NB):
        start_in(b, b)

    def group_body(t, _):
        for b in range(NB):
            g = NB * t + b

            @pl.when(g < NCHUNK)
            def _():
                @pl.when(g >= NB)
                def _():
                    wait_out(g - NB, b)

                wait_in(g, b)
                compute(b)
                start_out(g, b)

                @pl.when(g + NB < NCHUNK)
                def _():
                    start_in(g + NB, b)

        return 0

    lax.fori_loop(0, NGROUP, group_body, 0)
    for i in range(NB):
        g = NCHUNK - NB + i
        wait_out(g, g % NB)


def kernel(x, forward_shuffle_idx):
    x2 = x.reshape(ROWS, C)
    out = _shuffle_sc(x2, forward_shuffle_idx)
    return out.reshape(x.shape)


# speedup vs baseline: 1.1233x; 1.1233x over previous
import functools

import jax
import jax.numpy as jnp
from jax import lax
from jax.experimental import pallas as pl
from jax.experimental.pallas import tpu as pltpu
from jax.experimental.pallas import tpu_sc as plsc

L = 16
NC, NS = 2, 16
NW = NC * NS
C = 4096
ROWS = 4 * 4096
ROWS_PER_W = ROWS // NW
CHUNK = 4
NCHUNK = ROWS_PER_W // CHUNK
NB = 3
NGROUP = (NCHUNK + NB - 1) // NB

_mesh = plsc.VectorSubcoreMesh(
    core_axis_name="c", subcore_axis_name="s", num_cores=NC, num_subcores=NS
)


@functools.partial(
    pl.kernel,
    out_type=jax.ShapeDtypeStruct((ROWS, C), jnp.float32),
    mesh=_mesh,
    scratch_types=[
        pltpu.VMEM((C,), jnp.int32),
        pltpu.VMEM((CHUNK, C), jnp.float32),
        pltpu.VMEM((CHUNK, C), jnp.float32),
        pltpu.VMEM((CHUNK, C), jnp.float32),
        pltpu.VMEM((CHUNK, C), jnp.float32),
        pltpu.VMEM((CHUNK, C), jnp.float32),
        pltpu.VMEM((CHUNK, C), jnp.float32),
        pltpu.SemaphoreType.DMA,
        pltpu.SemaphoreType.DMA,
        pltpu.SemaphoreType.DMA,
        pltpu.SemaphoreType.DMA,
        pltpu.SemaphoreType.DMA,
        pltpu.SemaphoreType.DMA,
    ],
    compiler_params=pltpu.CompilerParams(needs_layout_passes=False),
)
def _shuffle_sc(x_hbm, idx_hbm, out_hbm, idx_v, in0, in1, in2,
                out0, out1, out2, isem0, isem1, isem2, osem0, osem1, osem2):
    wid = lax.axis_index("s") * NC + lax.axis_index("c")
    base = wid * ROWS_PER_W

    ins = (in0, in1, in2)
    outs = (out0, out1, out2)
    isems = (isem0, isem1, isem2)
    osems = (osem0, osem1, osem2)

    pltpu.sync_copy(idx_hbm, idx_v)

    def start_in(g, b):
        row0 = base + g * CHUNK
        pltpu.async_copy(x_hbm.at[pl.ds(row0, CHUNK), :], ins[b], isems[b])

    def wait_in(g, b):
        row0 = base + g * CHUNK
        pltpu.make_async_copy(
            x_hbm.at[pl.ds(row0, CHUNK), :], ins[b], isems[b]
        ).wait()

    def start_out(g, b):
        row0 = base + g * CHUNK
        pltpu.async_copy(outs[b], out_hbm.at[pl.ds(row0, CHUNK), :], osems[b])

    def wait_out(g, b):
        row0 = base + g * CHUNK
        pltpu.make_async_copy(
            outs[b], out_hbm.at[pl.ds(row0, CHUNK), :], osems[b]
        ).wait()

    def compute(b):
        in_v = ins[b]
        out_v = outs[b]

        @plsc.parallel_loop(0, C // L, unroll=8)
        def _(j):
            col = j * L
            idxv = idx_v[pl.ds(col, L)]
            for r in range(CHUNK):
                rvec = jnp.full((L,), r, jnp.int32)
                out_v[r, pl.ds(col, L)] = plsc.load_gather(in_v, [rvec, idxv])

    for b in range(NB):
        start_in(b, b)

    def group_body(t, _):
        for b in range(NB):
            g = NB * t + b

            @pl.when(g < NCHUNK)
            def _():
                @pl.when(g >= NB)
                def _():
                    wait_out(g - NB, b)

                wait_in(g, b)
                compute(b)
                start_out(g, b)

                @pl.when(g + NB < NCHUNK)
                def _():
                    start_in(g + NB, b)

        return 0

    lax.fori_loop(0, NGROUP, group_body, 0)
    for i in range(NB):
        g = NCHUNK - NB + i
        wait_out(g, g % NB)


def kernel(x, forward_shuffle_idx):
    x2 = x.reshape(ROWS, C)
    out = _shuffle_sc(x2, forward_shuffle_idx)
    return out.reshape(x.shape)
